# Initial kernel scaffold; baseline (speedup 1.0000x reference)
import jax
import jax.numpy as jnp
from jax.experimental import pallas as pl


def kernel(x, edge_index, edge_weight, conv1_W, conv1_b, conv2_W, conv2_b, W1, b1, W2, b2, ln_g, ln_b):
    raise NotImplementedError("write your pallas kernel here")


# speedup vs baseline: 1.9665x; 1.9665x over previous
import functools

import jax
import jax.numpy as jnp
from jax.experimental import pallas as pl
from jax.experimental.pallas import tpu as pltpu

N = 10000
D = 128
E = 160000

_SELU_ALPHA = 1.6732632423543772848170429916717
_SELU_SCALE = 1.0507009873554804934193349852946


def _selu(x):
    return _SELU_SCALE * jnp.where(x > 0, x, _SELU_ALPHA * (jnp.exp(x) - 1.0))


def _head_body(h_ref, w1_ref, b1_ref, w2_ref, b2_ref, g_ref, bb_ref, o_ref, *, B):
    h = h_ref[...]
    t = jnp.concatenate(
        [jax.lax.dot(h[b:b + 1, :], w1_ref[b]) for b in range(B)], axis=0)
    t = t + b1_ref[...]
    mu = jnp.mean(t, axis=-1, keepdims=True)
    var = jnp.mean((t - mu) ** 2, axis=-1, keepdims=True)
    t = (t - mu) * jax.lax.rsqrt(var + 1e-5) * g_ref[...] + bb_ref[...]
    o = jnp.concatenate(
        [jax.lax.dot(t[b:b + 1, :], w2_ref[b]) for b in range(B)], axis=0)
    o_ref[...] = o + b2_ref[...]


def _head(h, W1, b1, W2, b2, ln_g, ln_b, B=16):
    grid = (N // B,)
    return pl.pallas_call(
        functools.partial(_head_body, B=B),
        grid=grid,
        in_specs=[
            pl.BlockSpec((B, D), lambda i: (i, 0)),
            pl.BlockSpec((B, D, D), lambda i: (i, 0, 0)),
            pl.BlockSpec((B, D), lambda i: (i, 0)),
            pl.BlockSpec((B, D, D), lambda i: (i, 0, 0)),
            pl.BlockSpec((B, D), lambda i: (i, 0)),
            pl.BlockSpec((1, D), lambda i: (0, 0)),
            pl.BlockSpec((1, D), lambda i: (0, 0)),
        ],
        out_specs=pl.BlockSpec((B, D), lambda i: (i, 0)),
        out_shape=jax.ShapeDtypeStruct((N, D), jnp.float32),
    )(h, W1, b1, W2, b2, ln_g.reshape(1, D), ln_b.reshape(1, D))


def _gcn_conv(x, row, col, edge_weight, inv_deg, W, b):
    dis = jnp.sqrt(inv_deg)
    xw = x @ W
    y = xw * dis[:, None]
    msgs = y[row] * edge_weight[:, None]
    agg = jnp.zeros((N, D), jnp.float32).at[col].add(msgs)
    out = agg * dis[:, None] + xw * inv_deg[:, None]
    return out + b


def kernel(x, edge_index, edge_weight, conv1_W, conv1_b, conv2_W, conv2_b,
           W1, b1, W2, b2, ln_g, ln_b):
    row = edge_index[0]
    col = edge_index[1]
    deg = jnp.zeros((N,), jnp.float32).at[col].add(edge_weight) + 1.0
    inv_deg = 1.0 / deg
    h = _selu(_gcn_conv(x, row, col, edge_weight, inv_deg, conv1_W, conv1_b))
    h = _selu(_gcn_conv(h, row, col, edge_weight, inv_deg, conv2_W, conv2_b))
    return _head(h, W1, b1, W2, b2, ln_g, ln_b)


# speedup vs baseline: 3.8599x; 1.9628x over previous
import functools

import jax
import jax.numpy as jnp
from jax import lax
from jax.experimental import pallas as pl
from jax.experimental.pallas import tpu as pltpu
from jax.experimental.pallas import tpu_sc as plsc

N = 10000
D = 128
E = 160000

_NC = 2
_NS = 16
_TILES = _NC * _NS
_CH = 128
_NCH = 40
_EPT = _CH * _NCH
_EPAD = _TILES * _EPT
_NPAD = 10240

_SELU_ALPHA = 1.6732632423543772848170429916717
_SELU_SCALE = 1.0507009873554804934193349852946


def _selu(x):
    return _SELU_SCALE * jnp.where(x > 0, x, _SELU_ALPHA * (jnp.exp(x) - 1.0))


def _zero_fill_2d(buf, rows):
    z = jnp.zeros((16,), jnp.float32)
    for r in range(rows):
        for k in range(D // 16):
            buf[r, pl.ds(16 * k, 16)] = z


def _deg_call(col4, w4):
    mesh = plsc.VectorSubcoreMesh(core_axis_name="c", subcore_axis_name="s", num_cores=_NC, num_subcores=_NS)

    @functools.partial(
        pl.kernel,
        out_type=jax.ShapeDtypeStruct((_NC, 10, 1000), jnp.float32),
        mesh=mesh,
        scratch_types=[
            pltpu.VMEM((_NCH, _CH), jnp.int32),
            pltpu.VMEM((_NCH, _CH), jnp.float32),
            pltpu.VMEM((1000,), jnp.float32),
            pltpu.VMEM_SHARED((N,), jnp.float32),
        ],
        compiler_params=pltpu.CompilerParams(use_tc_tiling_on_sc=False, needs_layout_passes=False),
    )
    def deg_kernel(col_hbm, w_hbm, out_hbm, col_v, w_v, zbuf, acc):
        c = lax.axis_index("c")
        s = lax.axis_index("s")
        pltpu.sync_copy(col_hbm.at[c, s], col_v)
        pltpu.sync_copy(w_hbm.at[c, s], w_v)
        z = jnp.zeros((16,), jnp.float32)
        for k in range(1000 // 16):
            zbuf[pl.ds(16 * k, 16)] = z
        zbuf[pl.ds(984, 16)] = z

        @pl.when(s < 10)
        def _():
            pltpu.sync_copy(zbuf, acc.at[pl.ds(s * 1000, 1000)])

        plsc.subcore_barrier()

        def body(j, carry):
            pltpu.sync_copy(w_v.at[j], acc.at[col_v.at[j]], add=True)
            return carry

        lax.fori_loop(0, _NCH, body, 0)
        plsc.subcore_barrier()

        @pl.when(s < 10)
        def _():
            pltpu.sync_copy(acc.at[pl.ds(s * 1000, 1000)], out_hbm.at[c, s])

    return deg_kernel(col4, w4)


def _scale_rows(buf, w_v, j):
    jvec = jnp.full((16,), j, jnp.int32)
    for e in range(_CH):
        evec = jnp.full((16,), e, jnp.int32)
        wv = plsc.load_gather(w_v, [jvec, evec])
        for k in range(D // 16):
            sl = (e, pl.ds(16 * k, 16))
            buf[sl] = buf[sl] * wv


def _mp_call(y, row4, col4, w4):
    mesh = plsc.VectorSubcoreMesh(core_axis_name="c", subcore_axis_name="s", num_cores=_NC, num_subcores=_NS)

    @functools.partial(
        pl.kernel,
        out_type=jax.ShapeDtypeStruct((_NC, _NPAD, D), jnp.float32),
        mesh=mesh,
        scratch_types=[
            pltpu.VMEM((_NCH, _CH), jnp.int32),
            pltpu.VMEM((_NCH, _CH), jnp.int32),
            pltpu.VMEM((_NCH, _CH), jnp.float32),
            pltpu.VMEM((_CH, D), jnp.float32),
            pltpu.VMEM((_CH, D), jnp.float32),
            pltpu.VMEM_SHARED((_NPAD, D), jnp.float32),
            pltpu.SemaphoreType.DMA,
            pltpu.SemaphoreType.DMA,
        ],
        compiler_params=pltpu.CompilerParams(needs_layout_passes=False),
    )
    def mp_kernel(y_hbm, row_hbm, col_hbm, w_hbm, out_hbm,
                  row_v, col_v, w_v, buf0, buf1, acc, sem0, sem1):
        c = lax.axis_index("c")
        s = lax.axis_index("s")
        pltpu.sync_copy(row_hbm.at[c, s], row_v)
        pltpu.sync_copy(col_hbm.at[c, s], col_v)
        pltpu.sync_copy(w_hbm.at[c, s], w_v)

        _zero_fill_2d(buf0, _CH)
        for p in range(5):
            pltpu.sync_copy(buf0, acc.at[pl.ds(s * 640 + p * _CH, _CH)])
        plsc.subcore_barrier()

        pltpu.async_copy(y_hbm.at[row_v.at[0]], buf0, sem0)
        njj = _NCH // 2

        def body(jj, carry):
            j0 = 2 * jj
            pltpu.async_copy(y_hbm.at[row_v.at[j0 + 1]], buf1, sem1)
            pltpu.make_async_copy(y_hbm.at[row_v.at[j0]], buf0, sem0).wait()
            _scale_rows(buf0, w_v, j0)
            pltpu.sync_copy(buf0, acc.at[col_v.at[j0]], add=True)

            @pl.when(jj + 1 < njj)
            def _():
                pltpu.async_copy(y_hbm.at[row_v.at[j0 + 2]], buf0, sem0)

            pltpu.make_async_copy(y_hbm.at[row_v.at[j0 + 1]], buf1, sem1).wait()
            _scale_rows(buf1, w_v, j0 + 1)
            pltpu.sync_copy(buf1, acc.at[col_v.at[j0 + 1]], add=True)
            return carry

        lax.fori_loop(0, njj, body, 0)
        plsc.subcore_barrier()

        for p in range(5):
            r0 = s * 640 + p * _CH
            pltpu.sync_copy(acc.at[pl.ds(r0, _CH)],
                            out_hbm.at[c, pl.ds(r0, _CH)])

    return mp_kernel(y, row4, col4, w4)


def _prep_body(d0_ref, d1_ref, x_ref, w_ref, dis_ref, y_ref):
    deg = d0_ref[...] + d1_ref[...] + 1.0
    dis = jax.lax.rsqrt(deg)
    dis_ref[...] = dis
    y_ref[...] = jax.lax.dot(x_ref[...], w_ref[...]) * dis


def _prep(d0, d1, x, W, B=1000):
    grid = (N // B,)
    return pl.pallas_call(
        _prep_body,
        grid=grid,
        in_specs=[
            pl.BlockSpec((B, 1), lambda i: (i, 0)),
            pl.BlockSpec((B, 1), lambda i: (i, 0)),
            pl.BlockSpec((B, D), lambda i: (i, 0)),
            pl.BlockSpec((D, D), lambda i: (0, 0)),
        ],
        out_specs=[
            pl.BlockSpec((B, 1), lambda i: (i, 0)),
            pl.BlockSpec((B, D), lambda i: (i, 0)),
        ],
        out_shape=[
            jax.ShapeDtypeStruct((N, 1), jnp.float32),
            jax.ShapeDtypeStruct((N, D), jnp.float32),
        ],
    )(d0, d1, x, W)


def _combine_body(p0_ref, p1_ref, y_ref, dis_ref, b_ref, w_ref, y2_ref):
    dis = dis_ref[...]
    h = _selu(dis * (p0_ref[...] + p1_ref[...] + y_ref[...]) + b_ref[...])
    y2_ref[...] = jax.lax.dot(h, w_ref[...]) * dis


def _combine(p0, p1, y1, dis, b, W, B=1000):
    grid = (N // B,)
    return pl.pallas_call(
        _combine_body,
        grid=grid,
        in_specs=[
            pl.BlockSpec((B, D), lambda i: (i, 0)),
            pl.BlockSpec((B, D), lambda i: (i, 0)),
            pl.BlockSpec((B, D), lambda i: (i, 0)),
            pl.BlockSpec((B, 1), lambda i: (i, 0)),
            pl.BlockSpec((1, D), lambda i: (0, 0)),
            pl.BlockSpec((D, D), lambda i: (0, 0)),
        ],
        out_specs=pl.BlockSpec((B, D), lambda i: (i, 0)),
        out_shape=jax.ShapeDtypeStruct((N, D), jnp.float32),
    )(p0, p1, y1, dis, b, W)


def _head_body(q0_ref, q1_ref, y2_ref, dis_ref, bc_ref,
               w1_ref, b1_ref, w2_ref, b2_ref, g_ref, bb_ref, o_ref, *, B):
    dis = dis_ref[...]
    h = _selu(dis * (q0_ref[...] + q1_ref[...] + y2_ref[...]) + bc_ref[...])
    t = jnp.concatenate(
        [jax.lax.dot(h[b:b + 1, :], w1_ref[b]) for b in range(B)], axis=0)
    t = t + b1_ref[...]
    mu = jnp.mean(t, axis=-1, keepdims=True)
    var = jnp.mean((t - mu) ** 2, axis=-1, keepdims=True)
    t = (t - mu) * jax.lax.rsqrt(var + 1e-5) * g_ref[...] + bb_ref[...]
    o = jnp.concatenate(
        [jax.lax.dot(t[b:b + 1, :], w2_ref[b]) for b in range(B)], axis=0)
    o_ref[...] = o + b2_ref[...]


def _head(q0, q1, y2, dis, conv2_b, W1, b1, W2, b2, ln_g, ln_b, B=16):
    grid = (N // B,)
    return pl.pallas_call(
        functools.partial(_head_body, B=B),
        grid=grid,
        in_specs=[
            pl.BlockSpec((B, D), lambda i: (i, 0)),
            pl.BlockSpec((B, D), lambda i: (i, 0)),
            pl.BlockSpec((B, D), lambda i: (i, 0)),
            pl.BlockSpec((B, 1), lambda i: (i, 0)),
            pl.BlockSpec((1, D), lambda i: (0, 0)),
            pl.BlockSpec((B, D, D), lambda i: (i, 0, 0)),
            pl.BlockSpec((B, D), lambda i: (i, 0)),
            pl.BlockSpec((B, D, D), lambda i: (i, 0, 0)),
            pl.BlockSpec((B, D), lambda i: (i, 0)),
            pl.BlockSpec((1, D), lambda i: (0, 0)),
            pl.BlockSpec((1, D), lambda i: (0, 0)),
        ],
        out_specs=pl.BlockSpec((B, D), lambda i: (i, 0)),
        out_shape=jax.ShapeDtypeStruct((N, D), jnp.float32),
    )(q0, q1, y2, dis, conv2_b.reshape(1, D), W1, b1, W2, b2,
      ln_g.reshape(1, D), ln_b.reshape(1, D))


def kernel(x, edge_index, edge_weight, conv1_W, conv1_b, conv2_W, conv2_b,
           W1, b1, W2, b2, ln_g, ln_b):
    pad = _EPAD - E
    row4 = jnp.pad(edge_index[0], (0, pad)).reshape(_NC, _NS, _NCH, _CH)
    col4 = jnp.pad(edge_index[1], (0, pad)).reshape(_NC, _NS, _NCH, _CH)
    w4 = jnp.pad(edge_weight, (0, pad)).reshape(_NC, _NS, _NCH, _CH)

    deg2 = _deg_call(col4, w4).reshape(_NC, N)
    d0 = deg2[0].reshape(N, 1)
    d1 = deg2[1].reshape(N, 1)

    dis, y1 = _prep(d0, d1, x, conv1_W)

    p = _mp_call(y1, row4, col4, w4)
    y2 = _combine(p[0, :N], p[1, :N], y1, dis, conv1_b.reshape(1, D), conv2_W)

    q = _mp_call(y2, row4, col4, w4)
    return _head(q[0, :N], q[1, :N], y2, dis, conv2_b, W1, b1, W2, b2,
                 ln_g, ln_b)


# speedup vs baseline: 3.8918x; 1.0083x over previous
import functools

import jax
import jax.numpy as jnp
from jax import lax
from jax.experimental import pallas as pl
from jax.experimental.pallas import tpu as pltpu
from jax.experimental.pallas import tpu_sc as plsc

N = 10000
D = 128
E = 160000

_NC = 2
_NS = 16
_TILES = _NC * _NS
_CH = 128
_NCH = 40
_EPT = _CH * _NCH
_EPAD = _TILES * _EPT
_NPAD = 10240

_SELU_ALPHA = 1.6732632423543772848170429916717
_SELU_SCALE = 1.0507009873554804934193349852946


def _selu(x):
    return _SELU_SCALE * jnp.where(x > 0, x, _SELU_ALPHA * (jnp.exp(x) - 1.0))


def _zero_fill_2d(buf, rows):
    z = jnp.zeros((16,), jnp.float32)
    for r in range(rows):
        for k in range(D // 16):
            buf[r, pl.ds(16 * k, 16)] = z


def _deg_call(col4, w4):
    mesh = plsc.VectorSubcoreMesh(core_axis_name="c", subcore_axis_name="s", num_cores=_NC, num_subcores=_NS)

    @functools.partial(
        pl.kernel,
        out_type=jax.ShapeDtypeStruct((_NC, 10, 1000), jnp.float32),
        mesh=mesh,
        scratch_types=[
            pltpu.VMEM((_NCH, _CH), jnp.int32),
            pltpu.VMEM((_NCH, _CH), jnp.float32),
            pltpu.VMEM((1000,), jnp.float32),
            pltpu.VMEM_SHARED((N,), jnp.float32),
        ],
        compiler_params=pltpu.CompilerParams(use_tc_tiling_on_sc=False, needs_layout_passes=False),
    )
    def deg_kernel(col_hbm, w_hbm, out_hbm, col_v, w_v, zbuf, acc):
        c = lax.axis_index("c")
        s = lax.axis_index("s")
        pltpu.sync_copy(col_hbm.at[c, s], col_v)
        pltpu.sync_copy(w_hbm.at[c, s], w_v)
        z = jnp.zeros((16,), jnp.float32)
        for k in range(1000 // 16):
            zbuf[pl.ds(16 * k, 16)] = z
        zbuf[pl.ds(984, 16)] = z

        @pl.when(s < 10)
        def _():
            pltpu.sync_copy(zbuf, acc.at[pl.ds(s * 1000, 1000)])

        plsc.subcore_barrier()

        def body(j, carry):
            pltpu.sync_copy(w_v.at[j], acc.at[col_v.at[j]], add=True)
            return carry

        lax.fori_loop(0, _NCH, body, 0)
        plsc.subcore_barrier()

        @pl.when(s < 10)
        def _():
            pltpu.sync_copy(acc.at[pl.ds(s * 1000, 1000)], out_hbm.at[c, s])

    return deg_kernel(col4, w4)


def _scale_rows(buf, w_v, j):
    jvec = jnp.full((16,), j, jnp.int32)
    for e in range(_CH):
        evec = jnp.full((16,), e, jnp.int32)
        wv = plsc.load_gather(w_v, [jvec, evec])
        for k in range(D // 16):
            sl = (e, pl.ds(16 * k, 16))
            buf[sl] = buf[sl] * wv


def _mp_call(y, row4, col4, w4):
    mesh = plsc.VectorSubcoreMesh(core_axis_name="c", subcore_axis_name="s", num_cores=_NC, num_subcores=_NS)

    @functools.partial(
        pl.kernel,
        out_type=jax.ShapeDtypeStruct((_NC, _NPAD, D), jnp.float32),
        mesh=mesh,
        scratch_types=[
            pltpu.VMEM((_NCH, _CH), jnp.int32),
            pltpu.VMEM((_NCH, _CH), jnp.int32),
            pltpu.VMEM((_NCH, _CH), jnp.float32),
            pltpu.VMEM((_CH, D), jnp.float32),
            pltpu.VMEM((_CH, D), jnp.float32),
            pltpu.VMEM_SHARED((_NPAD, D), jnp.float32),
            pltpu.SemaphoreType.DMA,
            pltpu.SemaphoreType.DMA,
        ],
        compiler_params=pltpu.CompilerParams(needs_layout_passes=False),
    )
    def mp_kernel(y_hbm, row_hbm, col_hbm, w_hbm, out_hbm,
                  row_v, col_v, w_v, buf0, buf1, acc, sem0, sem1):
        c = lax.axis_index("c")
        s = lax.axis_index("s")
        pltpu.sync_copy(row_hbm.at[c, s], row_v)
        pltpu.sync_copy(col_hbm.at[c, s], col_v)
        pltpu.sync_copy(w_hbm.at[c, s], w_v)

        _zero_fill_2d(buf0, _CH)
        for p in range(5):
            pltpu.sync_copy(buf0, acc.at[pl.ds(s * 640 + p * _CH, _CH)])
        plsc.subcore_barrier()

        pltpu.async_copy(y_hbm.at[row_v.at[0]], buf0, sem0)
        njj = _NCH // 2

        def body(jj, carry):
            j0 = 2 * jj
            pltpu.async_copy(y_hbm.at[row_v.at[j0 + 1]], buf1, sem1)
            pltpu.make_async_copy(y_hbm.at[row_v.at[j0]], buf0, sem0).wait()
            _scale_rows(buf0, w_v, j0)
            pltpu.sync_copy(buf0, acc.at[col_v.at[j0]], add=True)

            @pl.when(jj + 1 < njj)
            def _():
                pltpu.async_copy(y_hbm.at[row_v.at[j0 + 2]], buf0, sem0)

            pltpu.make_async_copy(y_hbm.at[row_v.at[j0 + 1]], buf1, sem1).wait()
            _scale_rows(buf1, w_v, j0 + 1)
            pltpu.sync_copy(buf1, acc.at[col_v.at[j0 + 1]], add=True)
            return carry

        lax.fori_loop(0, njj, body, 0)
        plsc.subcore_barrier()

        for p in range(5):
            r0 = s * 640 + p * _CH
            pltpu.sync_copy(acc.at[pl.ds(r0, _CH)],
                            out_hbm.at[c, pl.ds(r0, _CH)])

    return mp_kernel(y, row4, col4, w4)


def _prep_body(d0_ref, d1_ref, x_ref, w_ref, dis_ref, y_ref):
    deg = d0_ref[...] + d1_ref[...] + 1.0
    dis = jax.lax.rsqrt(deg)
    dis_ref[...] = dis
    y_ref[...] = jax.lax.dot(x_ref[...], w_ref[...]) * dis


def _prep(d0, d1, x, W, B=1000):
    grid = (N // B,)
    return pl.pallas_call(
        _prep_body,
        grid=grid,
        in_specs=[
            pl.BlockSpec((B, 1), lambda i: (i, 0)),
            pl.BlockSpec((B, 1), lambda i: (i, 0)),
            pl.BlockSpec((B, D), lambda i: (i, 0)),
            pl.BlockSpec((D, D), lambda i: (0, 0)),
        ],
        out_specs=[
            pl.BlockSpec((B, 1), lambda i: (i, 0)),
            pl.BlockSpec((B, D), lambda i: (i, 0)),
        ],
        out_shape=[
            jax.ShapeDtypeStruct((N, 1), jnp.float32),
            jax.ShapeDtypeStruct((N, D), jnp.float32),
        ],
    )(d0, d1, x, W)


def _combine_body(p0_ref, p1_ref, y_ref, dis_ref, b_ref, w_ref, y2_ref):
    dis = dis_ref[...]
    h = _selu(dis * (p0_ref[...] + p1_ref[...] + y_ref[...]) + b_ref[...])
    y2_ref[...] = jax.lax.dot(h, w_ref[...]) * dis


def _combine(p0, p1, y1, dis, b, W, B=1000):
    grid = (N // B,)
    return pl.pallas_call(
        _combine_body,
        grid=grid,
        in_specs=[
            pl.BlockSpec((B, D), lambda i: (i, 0)),
            pl.BlockSpec((B, D), lambda i: (i, 0)),
            pl.BlockSpec((B, D), lambda i: (i, 0)),
            pl.BlockSpec((B, 1), lambda i: (i, 0)),
            pl.BlockSpec((1, D), lambda i: (0, 0)),
            pl.BlockSpec((D, D), lambda i: (0, 0)),
        ],
        out_specs=pl.BlockSpec((B, D), lambda i: (i, 0)),
        out_shape=jax.ShapeDtypeStruct((N, D), jnp.float32),
    )(p0, p1, y1, dis, b, W)


def _head_body(q0_ref, q1_ref, y2_ref, dis_ref, bc_ref,
               w1_ref, b1_ref, w2_ref, b2_ref, g_ref, bb_ref, o_ref, *, B):
    dis = dis_ref[...]
    h = _selu(dis * (q0_ref[...] + q1_ref[...] + y2_ref[...]) + bc_ref[...])
    colb = jax.lax.broadcasted_iota(jnp.int32, (B, B * D), 1) // D
    rowb = jax.lax.broadcasted_iota(jnp.int32, (B, B * D), 0)
    mask = colb == rowb
    hw = jnp.where(mask, jnp.concatenate([h] * B, axis=1), 0.0)
    t = jax.lax.dot(hw, w1_ref[...].reshape(B * D, D)) + b1_ref[...]
    mu = jnp.mean(t, axis=-1, keepdims=True)
    var = jnp.mean((t - mu) ** 2, axis=-1, keepdims=True)
    t = (t - mu) * jax.lax.rsqrt(var + 1e-5) * g_ref[...] + bb_ref[...]
    tw = jnp.where(mask, jnp.concatenate([t] * B, axis=1), 0.0)
    o_ref[...] = jax.lax.dot(tw, w2_ref[...].reshape(B * D, D)) + b2_ref[...]


def _head(q0, q1, y2, dis, conv2_b, W1, b1, W2, b2, ln_g, ln_b, B=16):
    grid = (N // B,)
    return pl.pallas_call(
        functools.partial(_head_body, B=B),
        grid=grid,
        in_specs=[
            pl.BlockSpec((B, D), lambda i: (i, 0)),
            pl.BlockSpec((B, D), lambda i: (i, 0)),
            pl.BlockSpec((B, D), lambda i: (i, 0)),
            pl.BlockSpec((B, 1), lambda i: (i, 0)),
            pl.BlockSpec((1, D), lambda i: (0, 0)),
            pl.BlockSpec((B, D, D), lambda i: (i, 0, 0)),
            pl.BlockSpec((B, D), lambda i: (i, 0)),
            pl.BlockSpec((B, D, D), lambda i: (i, 0, 0)),
            pl.BlockSpec((B, D), lambda i: (i, 0)),
            pl.BlockSpec((1, D), lambda i: (0, 0)),
            pl.BlockSpec((1, D), lambda i: (0, 0)),
        ],
        out_specs=pl.BlockSpec((B, D), lambda i: (i, 0)),
        out_shape=jax.ShapeDtypeStruct((N, D), jnp.float32),
    )(q0, q1, y2, dis, conv2_b.reshape(1, D), W1, b1, W2, b2,
      ln_g.reshape(1, D), ln_b.reshape(1, D))


def kernel(x, edge_index, edge_weight, conv1_W, conv1_b, conv2_W, conv2_b,
           W1, b1, W2, b2, ln_g, ln_b):
    pad = _EPAD - E
    row4 = jnp.pad(edge_index[0], (0, pad)).reshape(_NC, _NS, _NCH, _CH)
    col4 = jnp.pad(edge_index[1], (0, pad)).reshape(_NC, _NS, _NCH, _CH)
    w4 = jnp.pad(edge_weight, (0, pad)).reshape(_NC, _NS, _NCH, _CH)

    deg2 = _deg_call(col4, w4).reshape(_NC, N)
    d0 = deg2[0].reshape(N, 1)
    d1 = deg2[1].reshape(N, 1)

    dis, y1 = _prep(d0, d1, x, conv1_W)

    p = _mp_call(y1, row4, col4, w4)
    y2 = _combine(p[0, :N], p[1, :N], y1, dis, conv1_b.reshape(1, D), conv2_W)

    q = _mp_call(y2, row4, col4, w4)
    return _head(q[0, :N], q[1, :N], y2, dis, conv2_b, W1, b1, W2, b2,
                 ln_g, ln_b)


# speedup vs baseline: 4.6075x; 1.1839x over previous
import functools

import jax
import jax.numpy as jnp
from jax import lax
from jax.experimental import pallas as pl
from jax.experimental.pallas import tpu as pltpu
from jax.experimental.pallas import tpu_sc as plsc

N = 10000
D = 128
E = 160000

_NC = 2
_NS = 16
_TILES = _NC * _NS
_CH = 128
_NCH = 40
_EPT = _CH * _NCH
_EPAD = _TILES * _EPT
_NPAD = 10240

_SELU_ALPHA = 1.6732632423543772848170429916717
_SELU_SCALE = 1.0507009873554804934193349852946


def _selu(x):
    return _SELU_SCALE * jnp.where(x > 0, x, _SELU_ALPHA * (jnp.exp(x) - 1.0))


def _zero_fill_2d(buf, rows):
    z = jnp.zeros((16,), jnp.float32)
    for r in range(rows):
        for k in range(D // 16):
            buf[r, pl.ds(16 * k, 16)] = z


def _deg_call(col4, w4):
    mesh = plsc.VectorSubcoreMesh(core_axis_name="c", subcore_axis_name="s", num_cores=_NC, num_subcores=_NS)

    @functools.partial(
        pl.kernel,
        out_type=jax.ShapeDtypeStruct((_NC, 10, 1000), jnp.float32),
        mesh=mesh,
        scratch_types=[
            pltpu.VMEM((_NCH, _CH), jnp.int32),
            pltpu.VMEM((_NCH, _CH), jnp.float32),
            pltpu.VMEM((1000,), jnp.float32),
            pltpu.VMEM_SHARED((N,), jnp.float32),
        ],
        compiler_params=pltpu.CompilerParams(use_tc_tiling_on_sc=False, needs_layout_passes=False),
    )
    def deg_kernel(col_hbm, w_hbm, out_hbm, col_v, w_v, zbuf, acc):
        c = lax.axis_index("c")
        s = lax.axis_index("s")
        pltpu.sync_copy(col_hbm.at[c, s], col_v)
        pltpu.sync_copy(w_hbm.at[c, s], w_v)
        z = jnp.zeros((16,), jnp.float32)
        for k in range(1000 // 16):
            zbuf[pl.ds(16 * k, 16)] = z
        zbuf[pl.ds(984, 16)] = z

        @pl.when(s < 10)
        def _():
            pltpu.sync_copy(zbuf, acc.at[pl.ds(s * 1000, 1000)])

        plsc.subcore_barrier()

        def body(j, carry):
            pltpu.sync_copy(w_v.at[j], acc.at[col_v.at[j]], add=True)
            return carry

        lax.fori_loop(0, _NCH, body, 0)
        plsc.subcore_barrier()

        @pl.when(s < 10)
        def _():
            pltpu.sync_copy(acc.at[pl.ds(s * 1000, 1000)], out_hbm.at[c, s])

    return deg_kernel(col4, w4)


def _scale_rows(buf, w_v, j):
    jvec = jnp.full((16,), j, jnp.int32)
    for e in range(_CH):
        evec = jnp.full((16,), e, jnp.int32)
        wv = plsc.load_gather(w_v, [jvec, evec])
        for k in range(D // 16):
            sl = (e, pl.ds(16 * k, 16))
            buf[sl] = buf[sl] * wv


def _unpack_scale_half(gbuf, sbuf, w_v, j, h):
    jvec = jnp.full((16,), j, jnp.int32)
    for r in range(_CH // 2):
        e = 64 * h + r
        evec = jnp.full((16,), e, jnp.int32)
        wv = plsc.load_gather(w_v, [jvec, evec])
        for q in range(D // 32):
            v32 = gbuf[e, pl.ds(16 * q, 16)]
            vb = plsc.bitcast(v32, jnp.bfloat16)
            a, b = plsc.unpack(vb, format=plsc.PackFormat.INTERLEAVED)
            sbuf[r, pl.ds(32 * q, 16)] = a * wv
            sbuf[r, pl.ds(32 * q + 16, 16)] = b * wv


def _swz_bf16(y):
    y16 = y.astype(jnp.bfloat16)
    y16 = y16.reshape(N, 4, 2, 16).transpose(0, 1, 3, 2).reshape(N, D // 2, 2)
    return jax.lax.bitcast_convert_type(y16, jnp.int32)


def _mp_call(y, row4, col4, w4):
    mesh = plsc.VectorSubcoreMesh(core_axis_name="c", subcore_axis_name="s", num_cores=_NC, num_subcores=_NS)

    @functools.partial(
        pl.kernel,
        out_type=jax.ShapeDtypeStruct((_NC, _NPAD, D), jnp.float32),
        mesh=mesh,
        scratch_types=[
            pltpu.VMEM((_NCH, _CH), jnp.int32),
            pltpu.VMEM((_NCH * 2, _CH // 2), jnp.int32),
            pltpu.VMEM((_NCH, _CH), jnp.float32),
            pltpu.VMEM((_CH, D // 2), jnp.int32),
            pltpu.VMEM((_CH, D // 2), jnp.int32),
            pltpu.VMEM((_CH // 2, D), jnp.float32),
            pltpu.VMEM((_CH // 2, D), jnp.float32),
            pltpu.VMEM_SHARED((_NPAD, D), jnp.float32),
            pltpu.SemaphoreType.DMA,
            pltpu.SemaphoreType.DMA,
            pltpu.SemaphoreType.DMA,
            pltpu.SemaphoreType.DMA,
        ],
        compiler_params=pltpu.CompilerParams(use_tc_tiling_on_sc=False,
                                             needs_layout_passes=False),
    )
    def mp_kernel(y_hbm, row_hbm, col_hbm, w_hbm, out_hbm,
                  row_v, col_v, w_v, buf0, buf1, sbufa, sbufb, acc,
                  sem0, sem1, ssa, ssb):
        c = lax.axis_index("c")
        s = lax.axis_index("s")
        pltpu.sync_copy(row_hbm.at[c, s], row_v)
        pltpu.sync_copy(col_hbm.at[c, s], col_v)
        pltpu.sync_copy(w_hbm.at[c, s], w_v)

        _zero_fill_2d(sbufa, _CH // 2)
        for p in range(10):
            pltpu.sync_copy(sbufa, acc.at[pl.ds(s * 640 + p * 64, 64)])
        plsc.subcore_barrier()

        def _process(gbuf, j):
            for h, sb, ss in ((0, sbufa, ssa), (1, sbufb, ssb)):
                @pl.when(j > 0)
                def _():
                    pltpu.make_async_copy(sb, acc.at[pl.ds(0, 64)], ss).wait()

                _unpack_scale_half(gbuf, sb, w_v, j, h)
                pltpu.async_copy(sb, acc.at[col_v.at[2 * j + h]], ss,
                                 add=True)

        pltpu.async_copy(y_hbm.at[row_v.at[0]], buf0, sem0)
        njj = _NCH // 2

        def body(jj, carry):
            j0 = 2 * jj
            pltpu.async_copy(y_hbm.at[row_v.at[j0 + 1]], buf1, sem1)
            pltpu.make_async_copy(y_hbm.at[row_v.at[j0]], buf0, sem0).wait()
            _process(buf0, j0)

            @pl.when(jj + 1 < njj)
            def _():
                pltpu.async_copy(y_hbm.at[row_v.at[j0 + 2]], buf0, sem0)

            pltpu.make_async_copy(y_hbm.at[row_v.at[j0 + 1]], buf1, sem1).wait()
            _process(buf1, j0 + 1)
            return carry

        lax.fori_loop(0, njj, body, 0)
        pltpu.make_async_copy(sbufa, acc.at[pl.ds(0, 64)], ssa).wait()
        pltpu.make_async_copy(sbufb, acc.at[pl.ds(0, 64)], ssb).wait()
        plsc.subcore_barrier()

        for p in range(5):
            r0 = s * 640 + p * _CH
            pltpu.sync_copy(acc.at[pl.ds(r0, _CH)],
                            out_hbm.at[c, pl.ds(r0, _CH)])

    return mp_kernel(y, row4, col4, w4)


def _prep_body(d0_ref, d1_ref, x_ref, w_ref, dis_ref, y_ref):
    deg = d0_ref[...] + d1_ref[...] + 1.0
    dis = jax.lax.rsqrt(deg)
    dis_ref[...] = dis
    y_ref[...] = jax.lax.dot(x_ref[...], w_ref[...]) * dis


def _prep(d0, d1, x, W, B=1000):
    grid = (N // B,)
    return pl.pallas_call(
        _prep_body,
        grid=grid,
        in_specs=[
            pl.BlockSpec((B, 1), lambda i: (i, 0)),
            pl.BlockSpec((B, 1), lambda i: (i, 0)),
            pl.BlockSpec((B, D), lambda i: (i, 0)),
            pl.BlockSpec((D, D), lambda i: (0, 0)),
        ],
        out_specs=[
            pl.BlockSpec((B, 1), lambda i: (i, 0)),
            pl.BlockSpec((B, D), lambda i: (i, 0)),
        ],
        out_shape=[
            jax.ShapeDtypeStruct((N, 1), jnp.float32),
            jax.ShapeDtypeStruct((N, D), jnp.float32),
        ],
    )(d0, d1, x, W)


def _combine_body(p0_ref, p1_ref, y_ref, dis_ref, b_ref, w_ref, y2_ref):
    dis = dis_ref[...]
    h = _selu(dis * (p0_ref[...] + p1_ref[...] + y_ref[...]) + b_ref[...])
    y2_ref[...] = jax.lax.dot(h, w_ref[...]) * dis


def _combine(p0, p1, y1, dis, b, W, B=1000):
    grid = (N // B,)
    return pl.pallas_call(
        _combine_body,
        grid=grid,
        in_specs=[
            pl.BlockSpec((B, D), lambda i: (i, 0)),
            pl.BlockSpec((B, D), lambda i: (i, 0)),
            pl.BlockSpec((B, D), lambda i: (i, 0)),
            pl.BlockSpec((B, 1), lambda i: (i, 0)),
            pl.BlockSpec((1, D), lambda i: (0, 0)),
            pl.BlockSpec((D, D), lambda i: (0, 0)),
        ],
        out_specs=pl.BlockSpec((B, D), lambda i: (i, 0)),
        out_shape=jax.ShapeDtypeStruct((N, D), jnp.float32),
    )(p0, p1, y1, dis, b, W)


def _head_body(q0_ref, q1_ref, y2_ref, dis_ref, bc_ref,
               w1_ref, b1_ref, w2_ref, b2_ref, g_ref, bb_ref, o_ref, *, B):
    dis = dis_ref[...]
    h = _selu(dis * (q0_ref[...] + q1_ref[...] + y2_ref[...]) + bc_ref[...])
    colb = jax.lax.broadcasted_iota(jnp.int32, (B, B * D), 1) // D
    rowb = jax.lax.broadcasted_iota(jnp.int32, (B, B * D), 0)
    mask = colb == rowb
    hw = jnp.where(mask, jnp.concatenate([h] * B, axis=1), 0.0)
    t = jax.lax.dot(hw, w1_ref[...].reshape(B * D, D)) + b1_ref[...]
    mu = jnp.mean(t, axis=-1, keepdims=True)
    var = jnp.mean((t - mu) ** 2, axis=-1, keepdims=True)
    t = (t - mu) * jax.lax.rsqrt(var + 1e-5) * g_ref[...] + bb_ref[...]
    tw = jnp.where(mask, jnp.concatenate([t] * B, axis=1), 0.0)
    o_ref[...] = jax.lax.dot(tw, w2_ref[...].reshape(B * D, D)) + b2_ref[...]


def _head(q0, q1, y2, dis, conv2_b, W1, b1, W2, b2, ln_g, ln_b, B=16):
    grid = (N // B,)
    return pl.pallas_call(
        functools.partial(_head_body, B=B),
        grid=grid,
        in_specs=[
            pl.BlockSpec((B, D), lambda i: (i, 0)),
            pl.BlockSpec((B, D), lambda i: (i, 0)),
            pl.BlockSpec((B, D), lambda i: (i, 0)),
            pl.BlockSpec((B, 1), lambda i: (i, 0)),
            pl.BlockSpec((1, D), lambda i: (0, 0)),
            pl.BlockSpec((B, D, D), lambda i: (i, 0, 0)),
            pl.BlockSpec((B, D), lambda i: (i, 0)),
            pl.BlockSpec((B, D, D), lambda i: (i, 0, 0)),
            pl.BlockSpec((B, D), lambda i: (i, 0)),
            pl.BlockSpec((1, D), lambda i: (0, 0)),
            pl.BlockSpec((1, D), lambda i: (0, 0)),
        ],
        out_specs=pl.BlockSpec((B, D), lambda i: (i, 0)),
        out_shape=jax.ShapeDtypeStruct((N, D), jnp.float32),
    )(q0, q1, y2, dis, conv2_b.reshape(1, D), W1, b1, W2, b2,
      ln_g.reshape(1, D), ln_b.reshape(1, D))


def kernel(x, edge_index, edge_weight, conv1_W, conv1_b, conv2_W, conv2_b,
           W1, b1, W2, b2, ln_g, ln_b):
    pad = _EPAD - E
    row4 = jnp.pad(edge_index[0], (0, pad)).reshape(_NC, _NS, _NCH, _CH)
    colp = jnp.pad(edge_index[1], (0, pad))
    col4d = colp.reshape(_NC, _NS, _NCH, _CH)
    col4 = colp.reshape(_NC, _NS, _NCH * 2, _CH // 2)
    w4 = jnp.pad(edge_weight, (0, pad)).reshape(_NC, _NS, _NCH, _CH)

    deg2 = _deg_call(col4d, w4).reshape(_NC, N)
    d0 = deg2[0].reshape(N, 1)
    d1 = deg2[1].reshape(N, 1)

    dis, y1 = _prep(d0, d1, x, conv1_W)

    p = _mp_call(_swz_bf16(y1), row4, col4, w4)
    y2 = _combine(p[0, :N], p[1, :N], y1, dis, conv1_b.reshape(1, D), conv2_W)

    q = _mp_call(_swz_bf16(y2), row4, col4, w4)
    return _head(q[0, :N], q[1, :N], y2, dis, conv2_b, W1, b1, W2, b2,
                 ln_g, ln_b)
